# Initial kernel scaffold; baseline (speedup 1.0000x reference)
import jax
import jax.numpy as jnp
from jax.experimental import pallas as pl


def kernel(features, edge_list, W_heads, b_heads, a_heads, W_out, b_out, a_out):
    raise NotImplementedError("write your pallas kernel here")


# speedup vs baseline: 7.3876x; 7.3876x over previous
import jax
import jax.numpy as jnp
from jax import lax
from jax.experimental import pallas as pl
from jax.experimental.pallas import tpu as pltpu
from jax.experimental.pallas import tpu_sc as plsc

N = 10000
E = 320000
NFEAT = 128
NHID = 64
NHEAD = 4
NCLASS = 16
ALPHA = 0.2

G = 128
NCHUNK = 2560
E_PAD = NCHUNK * G
SUP = 16
W1COL = 136
W2COL = 32
NSC = 2
NTILE = 16
NRC = 79
TAIL = N - (NRC - 1) * G
TB = 1000
f32 = jnp.float32
i32 = jnp.int32


def _acc_chunks(s, fn_full, fn_tail):
    for k in range(5):
        j = s + NTILE * k
        off = pl.multiple_of(j * G, G)

        @pl.when(j < NRC - 1)
        def _():
            fn_full(off)

        @pl.when(j == NRC - 1)
        def _():
            fn_tail(off)


def _tc_prep1(feat_ref, w1_ref, acat_ref, htbl_ref, asrc_ref):
    x = feat_ref[...]
    h = jnp.dot(x, w1_ref[...], preferred_element_type=f32)
    al = jnp.dot(h, acat_ref[...], preferred_element_type=f32)
    asrc_ref[...] = al[:, 0:4]
    z = jnp.zeros((TB, W1COL - NFEAT - 2), f32)
    htbl_ref[0] = jnp.concatenate([h[:, 0:128], al[:, 4:5], al[:, 5:6], z], 1)
    htbl_ref[1] = jnp.concatenate([h[:, 128:256], al[:, 6:7], al[:, 7:8], z], 1)


def _sc_l1(htbl_hbm, ast_hbm, src_hbm, dst_hbm, acc_hbm,
           as_v, srcs_v, dsts_v, rowbuf, acc_s):
    c = lax.axis_index("c")
    s = lax.axis_index("s")
    pltpu.sync_copy(ast_hbm.at[2 * c], as_v.at[0])
    pltpu.sync_copy(ast_hbm.at[2 * c + 1], as_v.at[1])

    zv = jnp.zeros((16,), f32)

    @pl.loop(0, G)
    def _(r):
        for k in range(W1COL // 16):
            rowbuf[0, r, pl.ds(k * 16, 16)] = zv

    _acc_chunks(
        s,
        lambda off: pltpu.sync_copy(rowbuf.at[0], acc_s.at[pl.ds(off, G)]),
        lambda off: pltpu.sync_copy(rowbuf.at[0, 0:TAIL], acc_s.at[pl.ds(off, TAIL)]))
    plsc.subcore_barrier()

    iota16 = lax.broadcasted_iota(i32, (16,), 0)
    c128 = jnp.full((16,), NFEAT, i32)
    c129 = jnp.full((16,), NFEAT + 1, i32)
    coff = c * N

    @pl.loop(0, 10)
    def _(sup):
        base = s * 160 + sup * SUP
        pltpu.sync_copy(src_hbm.at[pl.ds(base, SUP)], srcs_v.at[0])
        pltpu.sync_copy(dst_hbm.at[pl.ds(base, SUP)], dsts_v.at[0])

        @pl.loop(0, SUP)
        def _(t):
            for g in range(8):
                dsts_v[0, t, pl.ds(g * 16, 16)] = (
                    dsts_v[0, t, pl.ds(g * 16, 16)] + coff)

        @pl.loop(0, SUP)
        def _(t):
            pltpu.sync_copy(htbl_hbm.at[dsts_v.at[0, t]], rowbuf.at[0])
            ebase = (base + t) * G
            for g in range(8):
                e16 = iota16 + (g * 16)
                idx_s = srcs_v[0, t, pl.ds(g * 16, 16)]
                ad_a = plsc.load_gather(rowbuf.at[0], [e16, c128])
                ad_b = plsc.load_gather(rowbuf.at[0], [e16, c129])
                as_a = plsc.load_gather(as_v, [jnp.zeros((16,), i32), idx_s])
                as_b = plsc.load_gather(as_v, [jnp.ones((16,), i32), idx_s])
                la = as_a + ad_a
                lb = as_b + ad_b
                wa = jnp.exp(jnp.where(la > 0, la, ALPHA * la))
                wb = jnp.exp(jnp.where(lb > 0, lb, ALPHA * lb))
                valid = (ebase + g * 16 + iota16) < E
                wa = jnp.where(valid, wa, 0.0)
                wb = jnp.where(valid, wb, 0.0)
                plsc.store_scatter(rowbuf.at[0], [e16, c128], wa)
                plsc.store_scatter(rowbuf.at[0], [e16, c129], wb)

            @pl.loop(0, G, unroll=4)
            def _(e):
                ef = jnp.full((16,), e, i32)
                wa = plsc.load_gather(rowbuf.at[0], [ef, c128])
                wb = plsc.load_gather(rowbuf.at[0], [ef, c129])
                for k in range(4):
                    rowbuf[0, e, pl.ds(k * 16, 16)] = (
                        rowbuf[0, e, pl.ds(k * 16, 16)] * wa)
                for k in range(4):
                    rowbuf[0, e, pl.ds(64 + k * 16, 16)] = (
                        rowbuf[0, e, pl.ds(64 + k * 16, 16)] * wb)

            pltpu.sync_copy(rowbuf.at[0], acc_s.at[srcs_v.at[0, t]], add=True)

    plsc.subcore_barrier()
    _acc_chunks(
        s,
        lambda off: pltpu.sync_copy(acc_s.at[pl.ds(off, G)],
                                    acc_hbm.at[c, pl.ds(off, G)]),
        lambda off: pltpu.sync_copy(acc_s.at[pl.ds(off, TAIL)],
                                    acc_hbm.at[c, pl.ds(off, TAIL)]))


def _tc_prep2(acc_ref, w2_ref, b2_ref, a2_ref, htbl2_ref, as2_ref):
    a = acc_ref[0]
    b = acc_ref[1]
    x0 = a[:, 0:64] / a[:, 128:129]
    x1 = a[:, 64:128] / a[:, 129:130]
    x2 = b[:, 0:64] / b[:, 128:129]
    x3 = b[:, 64:128] / b[:, 129:130]
    x = jnp.concatenate([x0, x1, x2, x3], axis=1)
    x = jnp.where(x > 0, x, jnp.exp(jnp.minimum(x, 0.0)) - 1.0)
    h2 = jnp.dot(x, w2_ref[...], preferred_element_type=f32) + b2_ref[...]
    al = jnp.dot(h2, a2_ref[...], preferred_element_type=f32)
    zb = jnp.zeros((TB, W2COL - NCLASS - 1), f32)
    htbl2_ref[...] = jnp.concatenate([h2, al[:, 1:2], zb], axis=1)
    as2_ref[...] = jnp.concatenate([al[:, 0:1], jnp.zeros((TB, 7), f32)], 1)


def _sc_l2(htbl2_hbm, as2_hbm, src_hbm, dst_hbm, acc_hbm,
           as_v, srcs_v, dsts_v, rowbuf, acc_s):
    c = lax.axis_index("c")
    s = lax.axis_index("s")
    wid = c * NTILE + s
    pltpu.sync_copy(as2_hbm, as_v)

    zv = jnp.zeros((16,), f32)

    @pl.loop(0, G)
    def _(r):
        for k in range(W2COL // 16):
            rowbuf[0, r, pl.ds(k * 16, 16)] = zv

    _acc_chunks(
        s,
        lambda off: pltpu.sync_copy(rowbuf.at[0], acc_s.at[pl.ds(off, G)]),
        lambda off: pltpu.sync_copy(rowbuf.at[0, 0:TAIL], acc_s.at[pl.ds(off, TAIL)]))
    plsc.subcore_barrier()

    iota16 = lax.broadcasted_iota(i32, (16,), 0)
    c16 = jnp.full((16,), NCLASS, i32)

    @pl.loop(0, 5)
    def _(sup):
        base = wid * 80 + sup * SUP
        pltpu.sync_copy(src_hbm.at[pl.ds(base, SUP)], srcs_v.at[0])
        pltpu.sync_copy(dst_hbm.at[pl.ds(base, SUP)], dsts_v.at[0])

        @pl.loop(0, SUP)
        def _(t):
            pltpu.sync_copy(htbl2_hbm.at[dsts_v.at[0, t]], rowbuf.at[0])
            ebase = (base + t) * G
            for g in range(8):
                e16 = iota16 + (g * 16)
                idx_s = srcs_v[0, t, pl.ds(g * 16, 16)]
                ad = plsc.load_gather(rowbuf.at[0], [e16, c16])
                a_s = plsc.load_gather(as_v, [idx_s])
                l = a_s + ad
                w = jnp.exp(jnp.where(l > 0, l, ALPHA * l))
                valid = (ebase + g * 16 + iota16) < E
                w = jnp.where(valid, w, 0.0)
                plsc.store_scatter(rowbuf.at[0], [e16, c16], w)

            @pl.loop(0, G, unroll=8)
            def _(e):
                ef = jnp.full((16,), e, i32)
                w = plsc.load_gather(rowbuf.at[0], [ef, c16])
                rowbuf[0, e, pl.ds(0, 16)] = rowbuf[0, e, pl.ds(0, 16)] * w

            pltpu.sync_copy(rowbuf.at[0], acc_s.at[srcs_v.at[0, t]], add=True)

    plsc.subcore_barrier()
    _acc_chunks(
        s,
        lambda off: pltpu.sync_copy(acc_s.at[pl.ds(off, G)],
                                    acc_hbm.at[c, pl.ds(off, G)]),
        lambda off: pltpu.sync_copy(acc_s.at[pl.ds(off, TAIL)],
                                    acc_hbm.at[c, pl.ds(off, TAIL)]))


def _tc_final(acc_ref, out_ref):
    t = acc_ref[0] + acc_ref[1]
    h = t[:, 0:NCLASS] / t[:, NCLASS:NCLASS + 1]
    z = h - jnp.max(h, axis=1, keepdims=True)
    out_ref[...] = z - jnp.log(jnp.sum(jnp.exp(z), axis=1, keepdims=True))


def _sc_mesh():
    return plsc.VectorSubcoreMesh(core_axis_name="c", subcore_axis_name="s",
                                  num_cores=NSC, num_subcores=NTILE)


def kernel(features, edge_list, W_heads, b_heads, a_heads, W_out, b_out, a_out):
    W1 = W_heads.reshape(NHEAD * NHID, NFEAT).T
    asrc = a_heads[:, 0, :NHID]
    adst = a_heads[:, 0, NHID:]
    eye = jnp.eye(NHEAD, dtype=f32)
    A_as = (eye[:, None, :] * asrc[:, :, None]).reshape(NHEAD * NHID, NHEAD)
    A_ad = (eye[:, None, :] * adst[:, :, None]).reshape(NHEAD * NHID, NHEAD)
    Acat = jnp.concatenate([A_as, A_ad], axis=1)
    pad = E_PAD - E
    src2d = jnp.pad(edge_list[0], (0, pad)).reshape(NCHUNK, G)
    dst2d = jnp.pad(edge_list[1], (0, pad)).reshape(NCHUNK, G)

    htbl, asrc_nm = pl.pallas_call(
        _tc_prep1,
        grid=(N // TB,),
        in_specs=[pl.BlockSpec((TB, NFEAT), lambda i: (i, 0)),
                  pl.BlockSpec((NFEAT, 256), lambda i: (0, 0)),
                  pl.BlockSpec((256, 8), lambda i: (0, 0))],
        out_specs=[pl.BlockSpec((2, TB, W1COL), lambda i: (0, i, 0)),
                   pl.BlockSpec((TB, 4), lambda i: (i, 0))],
        out_shape=[jax.ShapeDtypeStruct((NSC, N, W1COL), f32),
                   jax.ShapeDtypeStruct((N, 4), f32)],
    )(features, W1, Acat)
    htbl2d = htbl.reshape(NSC * N, W1COL)
    ast = asrc_nm.T

    acc1 = pl.kernel(
        _sc_l1,
        out_type=jax.ShapeDtypeStruct((NSC, N, W1COL), f32),
        mesh=_sc_mesh(),
        compiler_params=pltpu.CompilerParams(needs_layout_passes=False, use_tc_tiling_on_sc=False),
        scratch_types=[
            pltpu.VMEM((2, N), f32),
            pltpu.VMEM((1, SUP, G), i32),
            pltpu.VMEM((1, SUP, G), i32),
            pltpu.VMEM((1, G, W1COL), f32),
            pltpu.VMEM_SHARED((N, W1COL), f32),
        ],
    )(htbl2d, ast, src2d, dst2d)

    W2 = W_out.T
    a2cat = jnp.stack([a_out[0, :NCLASS], a_out[0, NCLASS:]], axis=1)
    htbl2, as2_nm = pl.pallas_call(
        _tc_prep2,
        grid=(N // TB,),
        in_specs=[pl.BlockSpec((2, TB, W1COL), lambda i: (0, i, 0)),
                  pl.BlockSpec((256, NCLASS), lambda i: (0, 0)),
                  pl.BlockSpec((1, NCLASS), lambda i: (0, 0)),
                  pl.BlockSpec((NCLASS, 2), lambda i: (0, 0))],
        out_specs=[pl.BlockSpec((TB, W2COL), lambda i: (i, 0)),
                   pl.BlockSpec((TB, 8), lambda i: (i, 0))],
        out_shape=[jax.ShapeDtypeStruct((N, W2COL), f32),
                   jax.ShapeDtypeStruct((N, 8), f32)],
    )(acc1, W2, b_out.reshape(1, NCLASS), a2cat)
    as2 = as2_nm[:, 0]

    acc2 = pl.kernel(
        _sc_l2,
        out_type=jax.ShapeDtypeStruct((NSC, N, W2COL), f32),
        mesh=_sc_mesh(),
        compiler_params=pltpu.CompilerParams(needs_layout_passes=False, use_tc_tiling_on_sc=False),
        scratch_types=[
            pltpu.VMEM((N,), f32),
            pltpu.VMEM((1, SUP, G), i32),
            pltpu.VMEM((1, SUP, G), i32),
            pltpu.VMEM((1, G, W2COL), f32),
            pltpu.VMEM_SHARED((N, W2COL), f32),
        ],
    )(htbl2, as2, src2d, dst2d)

    return pl.pallas_call(
        _tc_final,
        grid=(N // TB,),
        in_specs=[pl.BlockSpec((2, TB, W2COL), lambda i: (0, i, 0))],
        out_specs=pl.BlockSpec((TB, NCLASS), lambda i: (i, 0)),
        out_shape=jax.ShapeDtypeStruct((N, NCLASS), f32),
    )(acc2)


# speedup vs baseline: 9.6816x; 1.3105x over previous
import jax
import jax.numpy as jnp
from jax import lax
from jax.experimental import pallas as pl
from jax.experimental.pallas import tpu as pltpu
from jax.experimental.pallas import tpu_sc as plsc

N = 10000
E = 320000
NFEAT = 128
NHID = 64
NHEAD = 4
NCLASS = 16
ALPHA = 0.2

G = 128
NCHUNK = 2560
E_PAD = NCHUNK * G
SUP = 16
W1COL = 80
W2COL = 32
NSC = 2
NTILE = 16
NRC = 79
TAIL = N - (NRC - 1) * G
TB = 1000
NBUF = 4
LA = 3
f32 = jnp.float32
i32 = jnp.int32


def _acc_chunks(s, fn_full, fn_tail):
    for k in range(5):
        j = s + NTILE * k
        off = pl.multiple_of(j * G, G)

        @pl.when(j < NRC - 1)
        def _():
            fn_full(off)

        @pl.when(j == NRC - 1)
        def _():
            fn_tail(off)


def _zero_rowbuf0(rowbuf, width):
    zv = jnp.zeros((16,), f32)

    @pl.loop(0, G)
    def _(r):
        for k in range(width // 16):
            rowbuf[0, r, pl.ds(k * 16, 16)] = zv


def _edge_ring(nch, cbase, coff, tbl_hbm, src_hbm, dst_hbm,
               srcs_v, dsts_v, rowbuf, gsem, ssem, acc_s, attend_scale):
    def stage_sup(sup):
        db = lax.rem(sup, 2)
        gb = cbase + sup * SUP
        pltpu.sync_copy(src_hbm.at[pl.ds(gb, SUP)], srcs_v.at[db])
        pltpu.sync_copy(dst_hbm.at[pl.ds(gb, SUP)], dsts_v.at[db])

        if coff is not None:
            @pl.loop(0, SUP)
            def _(t):
                for g in range(8):
                    dsts_v[db, t, pl.ds(g * 16, 16)] = (
                        dsts_v[db, t, pl.ds(g * 16, 16)] + coff)

    def gather_desc(j, b):
        db = lax.rem(lax.div(j, SUP), 2)
        t = lax.rem(j, SUP)
        return pltpu.make_async_copy(
            tbl_hbm.at[dsts_v.at[db, t]], rowbuf.at[b], gsem.at[b])

    def scatter_desc(j, b):
        db = lax.rem(lax.div(j, SUP), 2)
        t = lax.rem(j, SUP)
        return pltpu.make_async_copy(
            rowbuf.at[b], acc_s.at[srcs_v.at[db, t]], ssem.at[b])

    stage_sup(0)
    for j0 in range(LA):
        gather_desc(j0, j0 % NBUF).start()

    @pl.loop(0, nch)
    def _(j):
        @pl.when(jnp.logical_and(lax.rem(j + LA, SUP) == 0, j + LA < nch))
        def _():
            stage_sup(lax.div(j + LA, SUP))

        @pl.when(j + LA < nch)
        def _():
            bn = lax.rem(j + LA, NBUF)

            @pl.when(j >= 1)
            def _():
                scatter_desc(j - 1, bn).wait()

            gather_desc(j + LA, bn).start()

        b = lax.rem(j, NBUF)
        gather_desc(j, b).wait()
        t = lax.rem(j, SUP)
        db = lax.rem(lax.div(j, SUP), 2)
        attend_scale(b, db, t, cbase + j)
        scatter_desc(j, b).start(add=True)

    for jd in range(nch - NBUF, nch):
        scatter_desc(jd, jd % NBUF).wait()


def _tc_prep1(feat_ref, w1_ref, acat_ref, htbl_ref, asrc_ref):
    x = feat_ref[...]
    h = jnp.dot(x, w1_ref[...], preferred_element_type=f32)
    al = jnp.dot(h, acat_ref[...], preferred_element_type=f32)
    asrc_ref[...] = al[:, 0:4]
    z = jnp.zeros((TB, W1COL - NHID - 1), f32)
    for k in range(NHEAD):
        htbl_ref[k] = jnp.concatenate(
            [h[:, k * NHID:(k + 1) * NHID], al[:, 4 + k:5 + k], z], 1)


def _sc_l1(htbl_hbm, ast_hbm, src_hbm, dst_hbm, acc_hbm,
           as_v, srcs_v, dsts_v, rowbuf, gsem, ssem, acc_s):
    c = lax.axis_index("c")
    s = lax.axis_index("s")
    iota16 = lax.broadcasted_iota(i32, (16,), 0)
    c64 = jnp.full((16,), NHID, i32)
    cbase = s * (NCHUNK // NTILE)

    for hh in range(2):
        hd = 2 * c + hh
        pltpu.sync_copy(ast_hbm.at[hd], as_v)
        _zero_rowbuf0(rowbuf, W1COL)
        _acc_chunks(
            s,
            lambda off: pltpu.sync_copy(rowbuf.at[0], acc_s.at[pl.ds(off, G)]),
            lambda off: pltpu.sync_copy(rowbuf.at[0, 0:TAIL],
                                        acc_s.at[pl.ds(off, TAIL)]))
        plsc.subcore_barrier()

        def attend_scale(b, db, t, jg):
            ebase = jg * G
            for g in range(8):
                e16 = iota16 + (g * 16)
                idx_s = srcs_v[db, t, pl.ds(g * 16, 16)]
                ad = plsc.load_gather(rowbuf.at[b], [e16, c64])
                a_s = plsc.load_gather(as_v, [idx_s])
                l = a_s + ad
                w = jnp.exp(jnp.where(l > 0, l, ALPHA * l))
                w = jnp.where((ebase + g * 16 + iota16) < E, w, 0.0)
                plsc.store_scatter(rowbuf.at[b], [e16, c64], w)

            @pl.loop(0, G, unroll=4)
            def _(e):
                ef = jnp.full((16,), e, i32)
                w = plsc.load_gather(rowbuf.at[b], [ef, c64])
                for k in range(4):
                    rowbuf[b, e, pl.ds(k * 16, 16)] = (
                        rowbuf[b, e, pl.ds(k * 16, 16)] * w)

        _edge_ring(NCHUNK // NTILE, cbase, hd * N, htbl_hbm, src_hbm, dst_hbm,
                   srcs_v, dsts_v, rowbuf, gsem, ssem, acc_s, attend_scale)
        plsc.subcore_barrier()
        _acc_chunks(
            s,
            lambda off: pltpu.sync_copy(acc_s.at[pl.ds(off, G)],
                                        acc_hbm.at[hd, pl.ds(off, G)]),
            lambda off: pltpu.sync_copy(acc_s.at[pl.ds(off, TAIL)],
                                        acc_hbm.at[hd, pl.ds(off, TAIL)]))
        if hh == 0:
            plsc.subcore_barrier()


def _tc_prep2(acc_ref, w2_ref, b2_ref, a2_ref, htbl2_ref, as2_ref):
    xs = [acc_ref[k][:, 0:NHID] / acc_ref[k][:, NHID:NHID + 1]
          for k in range(NHEAD)]
    x = jnp.concatenate(xs, axis=1)
    x = jnp.where(x > 0, x, jnp.exp(jnp.minimum(x, 0.0)) - 1.0)
    h2 = jnp.dot(x, w2_ref[...], preferred_element_type=f32) + b2_ref[...]
    al = jnp.dot(h2, a2_ref[...], preferred_element_type=f32)
    zb = jnp.zeros((TB, W2COL - NCLASS - 1), f32)
    htbl2_ref[...] = jnp.concatenate([h2, al[:, 1:2], zb], axis=1)
    as2_ref[...] = jnp.concatenate([al[:, 0:1], jnp.zeros((TB, 7), f32)], 1)


def _sc_l2(htbl2_hbm, as2_hbm, src_hbm, dst_hbm, acc_hbm,
           as_v, srcs_v, dsts_v, rowbuf, gsem, ssem, acc_s):
    c = lax.axis_index("c")
    s = lax.axis_index("s")
    wid = c * NTILE + s
    iota16 = lax.broadcasted_iota(i32, (16,), 0)
    c16 = jnp.full((16,), NCLASS, i32)
    pltpu.sync_copy(as2_hbm, as_v)
    _zero_rowbuf0(rowbuf, W2COL)
    _acc_chunks(
        s,
        lambda off: pltpu.sync_copy(rowbuf.at[0], acc_s.at[pl.ds(off, G)]),
        lambda off: pltpu.sync_copy(rowbuf.at[0, 0:TAIL],
                                    acc_s.at[pl.ds(off, TAIL)]))
    plsc.subcore_barrier()

    nch = NCHUNK // (NSC * NTILE)
    cbase = wid * nch

    def attend_scale(b, db, t, jg):
        ebase = jg * G
        for g in range(8):
            e16 = iota16 + (g * 16)
            idx_s = srcs_v[db, t, pl.ds(g * 16, 16)]
            ad = plsc.load_gather(rowbuf.at[b], [e16, c16])
            a_s = plsc.load_gather(as_v, [idx_s])
            l = a_s + ad
            w = jnp.exp(jnp.where(l > 0, l, ALPHA * l))
            w = jnp.where((ebase + g * 16 + iota16) < E, w, 0.0)
            plsc.store_scatter(rowbuf.at[b], [e16, c16], w)

        @pl.loop(0, G, unroll=8)
        def _(e):
            ef = jnp.full((16,), e, i32)
            w = plsc.load_gather(rowbuf.at[b], [ef, c16])
            rowbuf[b, e, pl.ds(0, 16)] = rowbuf[b, e, pl.ds(0, 16)] * w

    _edge_ring(nch, cbase, None, htbl2_hbm, src_hbm, dst_hbm,
               srcs_v, dsts_v, rowbuf, gsem, ssem, acc_s, attend_scale)
    plsc.subcore_barrier()
    _acc_chunks(
        s,
        lambda off: pltpu.sync_copy(acc_s.at[pl.ds(off, G)],
                                    acc_hbm.at[c, pl.ds(off, G)]),
        lambda off: pltpu.sync_copy(acc_s.at[pl.ds(off, TAIL)],
                                    acc_hbm.at[c, pl.ds(off, TAIL)]))


def _tc_final(acc_ref, out_ref):
    t = acc_ref[0] + acc_ref[1]
    h = t[:, 0:NCLASS] / t[:, NCLASS:NCLASS + 1]
    z = h - jnp.max(h, axis=1, keepdims=True)
    out_ref[...] = z - jnp.log(jnp.sum(jnp.exp(z), axis=1, keepdims=True))


def _sc_mesh():
    return plsc.VectorSubcoreMesh(core_axis_name="c", subcore_axis_name="s",
                                  num_cores=NSC, num_subcores=NTILE)


_SC_PARAMS = pltpu.CompilerParams(needs_layout_passes=False,
                                  use_tc_tiling_on_sc=False)


def kernel(features, edge_list, W_heads, b_heads, a_heads, W_out, b_out, a_out):
    W1 = W_heads.reshape(NHEAD * NHID, NFEAT).T
    asrc = a_heads[:, 0, :NHID]
    adst = a_heads[:, 0, NHID:]
    eye = jnp.eye(NHEAD, dtype=f32)
    A_as = (eye[:, None, :] * asrc[:, :, None]).reshape(NHEAD * NHID, NHEAD)
    A_ad = (eye[:, None, :] * adst[:, :, None]).reshape(NHEAD * NHID, NHEAD)
    Acat = jnp.concatenate([A_as, A_ad], axis=1)
    pad = E_PAD - E
    src2d = jnp.pad(edge_list[0], (0, pad)).reshape(NCHUNK, G)
    dst2d = jnp.pad(edge_list[1], (0, pad)).reshape(NCHUNK, G)

    htbl, asrc_nm = pl.pallas_call(
        _tc_prep1,
        grid=(N // TB,),
        in_specs=[pl.BlockSpec((TB, NFEAT), lambda i: (i, 0)),
                  pl.BlockSpec((NFEAT, 256), lambda i: (0, 0)),
                  pl.BlockSpec((256, 8), lambda i: (0, 0))],
        out_specs=[pl.BlockSpec((NHEAD, TB, W1COL), lambda i: (0, i, 0)),
                   pl.BlockSpec((TB, 4), lambda i: (i, 0))],
        out_shape=[jax.ShapeDtypeStruct((NHEAD, N, W1COL), f32),
                   jax.ShapeDtypeStruct((N, 4), f32)],
    )(features, W1, Acat)
    htbl2d = htbl.reshape(NHEAD * N, W1COL)
    ast = asrc_nm.T

    acc1 = pl.kernel(
        _sc_l1,
        out_type=jax.ShapeDtypeStruct((NHEAD, N, W1COL), f32),
        mesh=_sc_mesh(),
        compiler_params=_SC_PARAMS,
        scratch_types=[
            pltpu.VMEM((N,), f32),
            pltpu.VMEM((2, SUP, G), i32),
            pltpu.VMEM((2, SUP, G), i32),
            pltpu.VMEM((NBUF, G, W1COL), f32),
            pltpu.SemaphoreType.DMA((NBUF,)),
            pltpu.SemaphoreType.DMA((NBUF,)),
            pltpu.VMEM_SHARED((N, W1COL), f32),
        ],
    )(htbl2d, ast, src2d, dst2d)

    W2 = W_out.T
    a2cat = jnp.stack([a_out[0, :NCLASS], a_out[0, NCLASS:]], axis=1)
    htbl2, as2_nm = pl.pallas_call(
        _tc_prep2,
        grid=(N // TB,),
        in_specs=[pl.BlockSpec((NHEAD, TB, W1COL), lambda i: (0, i, 0)),
                  pl.BlockSpec((256, NCLASS), lambda i: (0, 0)),
                  pl.BlockSpec((1, NCLASS), lambda i: (0, 0)),
                  pl.BlockSpec((NCLASS, 2), lambda i: (0, 0))],
        out_specs=[pl.BlockSpec((TB, W2COL), lambda i: (i, 0)),
                   pl.BlockSpec((TB, 8), lambda i: (i, 0))],
        out_shape=[jax.ShapeDtypeStruct((N, W2COL), f32),
                   jax.ShapeDtypeStruct((N, 8), f32)],
    )(acc1, W2, b_out.reshape(1, NCLASS), a2cat)
    as2 = as2_nm[:, 0]

    acc2 = pl.kernel(
        _sc_l2,
        out_type=jax.ShapeDtypeStruct((NSC, N, W2COL), f32),
        mesh=_sc_mesh(),
        compiler_params=_SC_PARAMS,
        scratch_types=[
            pltpu.VMEM((N,), f32),
            pltpu.VMEM((2, SUP, G), i32),
            pltpu.VMEM((2, SUP, G), i32),
            pltpu.VMEM((NBUF, G, W2COL), f32),
            pltpu.SemaphoreType.DMA((NBUF,)),
            pltpu.SemaphoreType.DMA((NBUF,)),
            pltpu.VMEM_SHARED((N, W2COL), f32),
        ],
    )(htbl2, as2, src2d, dst2d)

    return pl.pallas_call(
        _tc_final,
        grid=(N // TB,),
        in_specs=[pl.BlockSpec((2, TB, W2COL), lambda i: (0, i, 0))],
        out_specs=pl.BlockSpec((TB, NCLASS), lambda i: (i, 0)),
        out_shape=jax.ShapeDtypeStruct((N, NCLASS), f32),
    )(acc2)


# speedup vs baseline: 11.7264x; 1.2112x over previous
import jax
import jax.numpy as jnp
from jax import lax
from jax.experimental import pallas as pl
from jax.experimental.pallas import tpu as pltpu
from jax.experimental.pallas import tpu_sc as plsc

N = 10000
E = 320000
NFEAT = 128
NHID = 64
NHEAD = 4
NCLASS = 16
ALPHA = 0.2

G = 128
NCHUNK = 2560
E_PAD = NCHUNK * G
SUP = 16
WS1 = 80
WS2 = 32
NSC = 2
NTILE = 16
NRC = 79
TAIL = N - (NRC - 1) * G
TB = 1000
NBUFG = 4
NBUFS = 2
LA = 3
f32 = jnp.float32
bf16 = jnp.bfloat16
i32 = jnp.int32


def _acc_chunks(s, fn_full, fn_tail):
    for k in range(5):
        j = s + NTILE * k
        off = pl.multiple_of(j * G, G)

        @pl.when(j < NRC - 1)
        def _():
            fn_full(off)

        @pl.when(j == NRC - 1)
        def _():
            fn_tail(off)


def _zero_buf0(buf, width):
    zv = jnp.zeros((16,), f32)

    @pl.loop(0, G)
    def _(r):
        for k in range(width // 16):
            buf[0, r, pl.ds(k * 16, 16)] = zv


def _edge_ring(nch, cbase, coff, tbl_hbm, src_hbm, dst_hbm,
               srcs_v, dsts_v, gbuf, sbuf, gsem, ssem, acc_s, attend_scale):
    def stage_sup(sup):
        db = lax.rem(sup, 2)
        gb = cbase + sup * SUP
        pltpu.sync_copy(src_hbm.at[pl.ds(gb, SUP)], srcs_v.at[db])
        pltpu.sync_copy(dst_hbm.at[pl.ds(gb, SUP)], dsts_v.at[db])

        if coff is not None:
            @pl.loop(0, SUP)
            def _(t):
                for g in range(8):
                    dsts_v[db, t, pl.ds(g * 16, 16)] = (
                        dsts_v[db, t, pl.ds(g * 16, 16)] + coff)

    def gather_desc(j, b):
        db = lax.rem(lax.div(j, SUP), 2)
        t = lax.rem(j, SUP)
        return pltpu.make_async_copy(
            tbl_hbm.at[dsts_v.at[db, t]], gbuf.at[b], gsem.at[b])

    def scatter_desc(j, b):
        db = lax.rem(lax.div(j, SUP), 2)
        t = lax.rem(j, SUP)
        return pltpu.make_async_copy(
            sbuf.at[b], acc_s.at[srcs_v.at[db, t]], ssem.at[b])

    stage_sup(0)
    for j0 in range(LA):
        gather_desc(j0, j0 % NBUFG).start()

    @pl.loop(0, nch)
    def _(j):
        @pl.when(jnp.logical_and(lax.rem(j + LA, SUP) == 0, j + LA < nch))
        def _():
            stage_sup(lax.div(j + LA, SUP))

        @pl.when(j + LA < nch)
        def _():
            gather_desc(j + LA, lax.rem(j + LA, NBUFG)).start()

        gather_desc(j, lax.rem(j, NBUFG)).wait()

        @pl.when(j >= NBUFS)
        def _():
            scatter_desc(j - NBUFS, lax.rem(j, NBUFS)).wait()

        t = lax.rem(j, SUP)
        db = lax.rem(lax.div(j, SUP), 2)
        attend_scale(lax.rem(j, NBUFG), lax.rem(j, NBUFS), db, t, cbase + j)
        scatter_desc(j, lax.rem(j, NBUFS)).start(add=True)

    for jd in range(nch - NBUFS, nch):
        scatter_desc(jd, jd % NBUFS).wait()


def _tc_prep1(feat_ref, w1_ref, acat_ref, h_ref, al_ref):
    x = feat_ref[...]
    h = jnp.dot(x, w1_ref[...], preferred_element_type=f32)
    al = jnp.dot(h, acat_ref[...], preferred_element_type=f32)
    al_ref[...] = al
    h_ref[...] = h


def _sc_l1(htbl_hbm, alt_hbm, src_hbm, dst_hbm, acc_hbm,
           as_v, ad_v, srcs_v, dsts_v, gbuf, sbuf, gsem, ssem, acc_s):
    c = lax.axis_index("c")
    s = lax.axis_index("s")
    iota16 = lax.broadcasted_iota(i32, (16,), 0)
    c64 = jnp.full((16,), NHID, i32)
    cbase = s * (NCHUNK // NTILE)

    for hh in range(2):
        hd = 2 * c + hh
        coff = hd * N
        pltpu.sync_copy(alt_hbm.at[hd], as_v)
        pltpu.sync_copy(alt_hbm.at[NHEAD + hd], ad_v)
        _zero_buf0(sbuf, WS1)
        _acc_chunks(
            s,
            lambda off: pltpu.sync_copy(sbuf.at[0], acc_s.at[pl.ds(off, G)]),
            lambda off: pltpu.sync_copy(sbuf.at[0, 0:TAIL],
                                        acc_s.at[pl.ds(off, TAIL)]))
        plsc.subcore_barrier()

        def attend_scale(gb, sb, db, t, jg):
            ebase = jg * G
            for g in range(8):
                e16 = iota16 + (g * 16)
                idx_s = srcs_v[db, t, pl.ds(g * 16, 16)]
                idx_d = dsts_v[db, t, pl.ds(g * 16, 16)] - coff
                a_s = plsc.load_gather(as_v, [idx_s])
                a_d = plsc.load_gather(ad_v, [idx_d])
                l = a_s + a_d
                w = jnp.exp(jnp.where(l > 0, l, ALPHA * l))
                w = jnp.where((ebase + g * 16 + iota16) < E, w, 0.0)
                plsc.store_scatter(sbuf.at[sb], [e16, c64], w)

            @pl.loop(0, G, unroll=4)
            def _(e):
                ef = jnp.full((16,), e, i32)
                w = plsc.load_gather(sbuf.at[sb], [ef, c64])
                for k in range(2):
                    x2 = gbuf[gb, e, pl.ds(k * 32, 32)]
                    lo, hi = plsc.unpack(x2, format=plsc.PackFormat.INTERLEAVED)
                    sbuf[sb, e, pl.ds(k * 32, 16)] = lo * w
                    sbuf[sb, e, pl.ds(k * 32 + 16, 16)] = hi * w

        _edge_ring(NCHUNK // NTILE, cbase, coff, htbl_hbm, src_hbm, dst_hbm,
                   srcs_v, dsts_v, gbuf, sbuf, gsem, ssem, acc_s, attend_scale)
        plsc.subcore_barrier()
        _acc_chunks(
            s,
            lambda off: pltpu.sync_copy(acc_s.at[pl.ds(off, G)],
                                        acc_hbm.at[hd, pl.ds(off, G)]),
            lambda off: pltpu.sync_copy(acc_s.at[pl.ds(off, TAIL)],
                                        acc_hbm.at[hd, pl.ds(off, TAIL)]))
        if hh == 0:
            plsc.subcore_barrier()


def _tc_prep2(acc_ref, w2_ref, b2_ref, a2_ref, htbl2_ref, al2_ref):
    xs = [acc_ref[k][:, 0:NHID] / acc_ref[k][:, NHID:NHID + 1]
          for k in range(NHEAD)]
    x = jnp.concatenate(xs, axis=1)
    x = jnp.where(x > 0, x, jnp.exp(jnp.minimum(x, 0.0)) - 1.0)
    h2 = jnp.dot(x, w2_ref[...], preferred_element_type=f32) + b2_ref[...]
    al = jnp.dot(h2, a2_ref[...], preferred_element_type=f32)
    htbl2_ref[...] = h2
    al2_ref[...] = jnp.concatenate([al, jnp.zeros((TB, 6), f32)], 1)


def _sc_l2(htbl2_hbm, al2t_hbm, src_hbm, dst_hbm, acc_hbm,
           as_v, ad_v, srcs_v, dsts_v, gbuf, sbuf, gsem, ssem, acc_s):
    c = lax.axis_index("c")
    s = lax.axis_index("s")
    wid = c * NTILE + s
    iota16 = lax.broadcasted_iota(i32, (16,), 0)
    c16 = jnp.full((16,), NCLASS, i32)
    pltpu.sync_copy(al2t_hbm.at[0], as_v)
    pltpu.sync_copy(al2t_hbm.at[1], ad_v)
    _zero_buf0(sbuf, WS2)
    _acc_chunks(
        s,
        lambda off: pltpu.sync_copy(sbuf.at[0], acc_s.at[pl.ds(off, G)]),
        lambda off: pltpu.sync_copy(sbuf.at[0, 0:TAIL],
                                    acc_s.at[pl.ds(off, TAIL)]))
    plsc.subcore_barrier()

    nch = NCHUNK // (NSC * NTILE)
    cbase = wid * nch

    def attend_scale(gb, sb, db, t, jg):
        ebase = jg * G
        for g in range(8):
            e16 = iota16 + (g * 16)
            idx_s = srcs_v[db, t, pl.ds(g * 16, 16)]
            idx_d = dsts_v[db, t, pl.ds(g * 16, 16)]
            a_s = plsc.load_gather(as_v, [idx_s])
            a_d = plsc.load_gather(ad_v, [idx_d])
            l = a_s + a_d
            w = jnp.exp(jnp.where(l > 0, l, ALPHA * l))
            w = jnp.where((ebase + g * 16 + iota16) < E, w, 0.0)
            plsc.store_scatter(sbuf.at[sb], [e16, c16], w)

        @pl.loop(0, G, unroll=8)
        def _(e):
            ef = jnp.full((16,), e, i32)
            w = plsc.load_gather(sbuf.at[sb], [ef, c16])
            sbuf[sb, e, pl.ds(0, 16)] = gbuf[gb, e, pl.ds(0, 16)] * w

    _edge_ring(nch, cbase, None, htbl2_hbm, src_hbm, dst_hbm,
               srcs_v, dsts_v, gbuf, sbuf, gsem, ssem, acc_s, attend_scale)
    plsc.subcore_barrier()
    _acc_chunks(
        s,
        lambda off: pltpu.sync_copy(acc_s.at[pl.ds(off, G)],
                                    acc_hbm.at[c, pl.ds(off, G)]),
        lambda off: pltpu.sync_copy(acc_s.at[pl.ds(off, TAIL)],
                                    acc_hbm.at[c, pl.ds(off, TAIL)]))


def _tc_final(acc_ref, out_ref):
    t = acc_ref[0] + acc_ref[1]
    h = t[:, 0:NCLASS] / t[:, NCLASS:NCLASS + 1]
    z = h - jnp.max(h, axis=1, keepdims=True)
    out_ref[...] = z - jnp.log(jnp.sum(jnp.exp(z), axis=1, keepdims=True))


def _sc_mesh():
    return plsc.VectorSubcoreMesh(core_axis_name="c", subcore_axis_name="s",
                                  num_cores=NSC, num_subcores=NTILE)


_SC_PARAMS = pltpu.CompilerParams(needs_layout_passes=False,
                                  use_tc_tiling_on_sc=False)


def kernel(features, edge_list, W_heads, b_heads, a_heads, W_out, b_out, a_out):
    W1 = W_heads.reshape(NHEAD * NHID, NFEAT).T
    asrc = a_heads[:, 0, :NHID]
    adst = a_heads[:, 0, NHID:]
    eye = jnp.eye(NHEAD, dtype=f32)
    A_as = (eye[:, None, :] * asrc[:, :, None]).reshape(NHEAD * NHID, NHEAD)
    A_ad = (eye[:, None, :] * adst[:, :, None]).reshape(NHEAD * NHID, NHEAD)
    Acat = jnp.concatenate([A_as, A_ad], axis=1)
    pad = E_PAD - E
    src2d = jnp.pad(edge_list[0], (0, pad)).reshape(NCHUNK, G)
    dst2d = jnp.pad(edge_list[1], (0, pad)).reshape(NCHUNK, G)

    h_all, al_nm = pl.pallas_call(
        _tc_prep1,
        grid=(N // TB,),
        in_specs=[pl.BlockSpec((TB, NFEAT), lambda i: (i, 0)),
                  pl.BlockSpec((NFEAT, 256), lambda i: (0, 0)),
                  pl.BlockSpec((256, 8), lambda i: (0, 0))],
        out_specs=[pl.BlockSpec((TB, 256), lambda i: (i, 0)),
                   pl.BlockSpec((TB, 8), lambda i: (i, 0))],
        out_shape=[jax.ShapeDtypeStruct((N, 256), f32),
                   jax.ShapeDtypeStruct((N, 8), f32)],
    )(features, W1, Acat)
    htbl2d = (h_all.reshape(N, NHEAD, 2, 2, 16)
              .transpose(1, 0, 2, 4, 3)
              .reshape(NHEAD * N, NHID).astype(bf16))
    alt = al_nm.T

    acc1 = pl.kernel(
        _sc_l1,
        out_type=jax.ShapeDtypeStruct((NHEAD, N, WS1), f32),
        mesh=_sc_mesh(),
        compiler_params=_SC_PARAMS,
        scratch_types=[
            pltpu.VMEM((N,), f32),
            pltpu.VMEM((N,), f32),
            pltpu.VMEM((2, SUP, G), i32),
            pltpu.VMEM((2, SUP, G), i32),
            pltpu.VMEM((NBUFG, G, NHID), bf16),
            pltpu.VMEM((NBUFS, G, WS1), f32),
            pltpu.SemaphoreType.DMA((NBUFG,)),
            pltpu.SemaphoreType.DMA((NBUFS,)),
            pltpu.VMEM_SHARED((N, WS1), f32),
        ],
    )(htbl2d, alt, src2d, dst2d)

    W2 = W_out.T
    a2cat = jnp.stack([a_out[0, :NCLASS], a_out[0, NCLASS:]], axis=1)
    htbl2, al2_nm = pl.pallas_call(
        _tc_prep2,
        grid=(N // TB,),
        in_specs=[pl.BlockSpec((NHEAD, TB, WS1), lambda i: (0, i, 0)),
                  pl.BlockSpec((256, NCLASS), lambda i: (0, 0)),
                  pl.BlockSpec((1, NCLASS), lambda i: (0, 0)),
                  pl.BlockSpec((NCLASS, 2), lambda i: (0, 0))],
        out_specs=[pl.BlockSpec((TB, NCLASS), lambda i: (i, 0)),
                   pl.BlockSpec((TB, 8), lambda i: (i, 0))],
        out_shape=[jax.ShapeDtypeStruct((N, NCLASS), f32),
                   jax.ShapeDtypeStruct((N, 8), f32)],
    )(acc1, W2, b_out.reshape(1, NCLASS), a2cat)
    al2t = al2_nm[:, 0:2].T

    acc2 = pl.kernel(
        _sc_l2,
        out_type=jax.ShapeDtypeStruct((NSC, N, WS2), f32),
        mesh=_sc_mesh(),
        compiler_params=_SC_PARAMS,
        scratch_types=[
            pltpu.VMEM((N,), f32),
            pltpu.VMEM((N,), f32),
            pltpu.VMEM((2, SUP, G), i32),
            pltpu.VMEM((2, SUP, G), i32),
            pltpu.VMEM((NBUFG, G, NCLASS), f32),
            pltpu.VMEM((NBUFS, G, WS2), f32),
            pltpu.SemaphoreType.DMA((NBUFG,)),
            pltpu.SemaphoreType.DMA((NBUFS,)),
            pltpu.VMEM_SHARED((N, WS2), f32),
        ],
    )(htbl2, al2t, src2d, dst2d)

    return pl.pallas_call(
        _tc_final,
        grid=(N // TB,),
        in_specs=[pl.BlockSpec((2, TB, WS2), lambda i: (0, i, 0))],
        out_specs=pl.BlockSpec((TB, NCLASS), lambda i: (i, 0)),
        out_shape=jax.ShapeDtypeStruct((N, NCLASS), f32),
    )(acc2)


# speedup vs baseline: 11.7296x; 1.0003x over previous
import jax
import jax.numpy as jnp
from jax import lax
from jax.experimental import pallas as pl
from jax.experimental.pallas import tpu as pltpu
from jax.experimental.pallas import tpu_sc as plsc

N = 10000
E = 320000
NFEAT = 128
NHID = 64
NHEAD = 4
NCLASS = 16
ALPHA = 0.2

G = 128
NCHUNK = 2560
E_PAD = NCHUNK * G
SUP = 16
WS1 = 68
WS2 = 20
NSC = 2
NTILE = 16
NRC = 79
TAIL = N - (NRC - 1) * G
TB = 1000
NBUFG = 4
NBUFS = 2
LA = 3
f32 = jnp.float32
bf16 = jnp.bfloat16
i32 = jnp.int32


def _acc_chunks(s, fn_full, fn_tail):
    for k in range(5):
        j = s + NTILE * k
        off = pl.multiple_of(j * G, G)

        @pl.when(j < NRC - 1)
        def _():
            fn_full(off)

        @pl.when(j == NRC - 1)
        def _():
            fn_tail(off)


def _zero_buf0(buf, width):
    zv = jnp.zeros((16,), f32)
    offs = [k * 16 for k in range(width // 16)]
    if width % 16:
        offs.append(width - 16)

    @pl.loop(0, G)
    def _(r):
        for o in offs:
            buf[0, r, pl.ds(o, 16)] = zv


def _edge_ring(nch, cbase, coff, tbl_hbm, src_hbm, dst_hbm,
               srcs_v, dsts_v, gbuf, sbuf, gsem, ssem, acc_s, attend_scale):
    def stage_sup(sup):
        db = lax.rem(sup, 2)
        gb = cbase + sup * SUP
        pltpu.sync_copy(src_hbm.at[pl.ds(gb, SUP)], srcs_v.at[db])
        pltpu.sync_copy(dst_hbm.at[pl.ds(gb, SUP)], dsts_v.at[db])

        if coff is not None:
            @pl.loop(0, SUP)
            def _(t):
                for g in range(8):
                    dsts_v[db, t, pl.ds(g * 16, 16)] = (
                        dsts_v[db, t, pl.ds(g * 16, 16)] + coff)

    def gather_desc(j, b):
        db = lax.rem(lax.div(j, SUP), 2)
        t = lax.rem(j, SUP)
        return pltpu.make_async_copy(
            tbl_hbm.at[dsts_v.at[db, t]], gbuf.at[b], gsem.at[b])

    def scatter_desc(j, b):
        db = lax.rem(lax.div(j, SUP), 2)
        t = lax.rem(j, SUP)
        return pltpu.make_async_copy(
            sbuf.at[b], acc_s.at[srcs_v.at[db, t]], ssem.at[b])

    stage_sup(0)
    for j0 in range(LA):
        gather_desc(j0, j0 % NBUFG).start()

    @pl.loop(0, nch)
    def _(j):
        @pl.when(jnp.logical_and(lax.rem(j + LA, SUP) == 0, j + LA < nch))
        def _():
            stage_sup(lax.div(j + LA, SUP))

        @pl.when(j + LA < nch)
        def _():
            gather_desc(j + LA, lax.rem(j + LA, NBUFG)).start()

        gather_desc(j, lax.rem(j, NBUFG)).wait()

        @pl.when(j >= NBUFS)
        def _():
            scatter_desc(j - NBUFS, lax.rem(j, NBUFS)).wait()

        t = lax.rem(j, SUP)
        db = lax.rem(lax.div(j, SUP), 2)
        attend_scale(lax.rem(j, NBUFG), lax.rem(j, NBUFS), db, t, cbase + j)
        scatter_desc(j, lax.rem(j, NBUFS)).start(add=True)

    for jd in range(nch - NBUFS, nch):
        scatter_desc(jd, jd % NBUFS).wait()


def _tc_prep1(feat_ref, w1_ref, acat_ref, h_ref, al_ref):
    x = feat_ref[...]
    h = jnp.dot(x, w1_ref[...], preferred_element_type=f32)
    al = jnp.dot(h, acat_ref[...], preferred_element_type=f32)
    al_ref[...] = al
    h_ref[...] = h


def _sc_l1(htbl_hbm, alt_hbm, src_hbm, dst_hbm, acc_hbm,
           as_v, ad_v, srcs_v, dsts_v, gbuf, sbuf, gsem, ssem, acc_s):
    c = lax.axis_index("c")
    s = lax.axis_index("s")
    iota16 = lax.broadcasted_iota(i32, (16,), 0)
    c64 = jnp.full((16,), NHID, i32)
    cbase = s * (NCHUNK // NTILE)

    for hh in range(2):
        hd = 2 * c + hh
        coff = hd * N
        pltpu.sync_copy(alt_hbm.at[hd], as_v)
        pltpu.sync_copy(alt_hbm.at[NHEAD + hd], ad_v)
        _zero_buf0(sbuf, WS1)
        _acc_chunks(
            s,
            lambda off: pltpu.sync_copy(sbuf.at[0], acc_s.at[pl.ds(off, G)]),
            lambda off: pltpu.sync_copy(sbuf.at[0, 0:TAIL],
                                        acc_s.at[pl.ds(off, TAIL)]))
        plsc.subcore_barrier()

        def attend_scale(gb, sb, db, t, jg):
            ebase = jg * G
            for g in range(8):
                e16 = iota16 + (g * 16)
                idx_s = srcs_v[db, t, pl.ds(g * 16, 16)]
                idx_d = dsts_v[db, t, pl.ds(g * 16, 16)] - coff
                a_s = plsc.load_gather(as_v, [idx_s])
                a_d = plsc.load_gather(ad_v, [idx_d])
                l = a_s + a_d
                w = jnp.exp(jnp.where(l > 0, l, ALPHA * l))
                w = jnp.where((ebase + g * 16 + iota16) < E, w, 0.0)
                plsc.store_scatter(sbuf.at[sb], [e16, c64], w)

            @pl.loop(0, G, unroll=4)
            def _(e):
                ef = jnp.full((16,), e, i32)
                w = plsc.load_gather(sbuf.at[sb], [ef, c64])
                for k in range(2):
                    x2 = gbuf[gb, e, pl.ds(k * 32, 32)]
                    lo, hi = plsc.unpack(x2, format=plsc.PackFormat.INTERLEAVED)
                    sbuf[sb, e, pl.ds(k * 32, 16)] = lo * w
                    sbuf[sb, e, pl.ds(k * 32 + 16, 16)] = hi * w

        _edge_ring(NCHUNK // NTILE, cbase, coff, htbl_hbm, src_hbm, dst_hbm,
                   srcs_v, dsts_v, gbuf, sbuf, gsem, ssem, acc_s, attend_scale)
        plsc.subcore_barrier()
        _acc_chunks(
            s,
            lambda off: pltpu.sync_copy(acc_s.at[pl.ds(off, G)],
                                        acc_hbm.at[hd, pl.ds(off, G)]),
            lambda off: pltpu.sync_copy(acc_s.at[pl.ds(off, TAIL)],
                                        acc_hbm.at[hd, pl.ds(off, TAIL)]))
        if hh == 0:
            plsc.subcore_barrier()


def _tc_prep2(acc_ref, w2_ref, b2_ref, a2_ref, htbl2_ref, al2_ref):
    xs = [acc_ref[k][:, 0:NHID] / acc_ref[k][:, NHID:NHID + 1]
          for k in range(NHEAD)]
    x = jnp.concatenate(xs, axis=1)
    x = jnp.where(x > 0, x, jnp.exp(jnp.minimum(x, 0.0)) - 1.0)
    h2 = jnp.dot(x, w2_ref[...], preferred_element_type=f32) + b2_ref[...]
    al = jnp.dot(h2, a2_ref[...], preferred_element_type=f32)
    htbl2_ref[...] = h2
    al2_ref[...] = jnp.concatenate([al, jnp.zeros((TB, 6), f32)], 1)


def _sc_l2(htbl2_hbm, al2t_hbm, src_hbm, dst_hbm, acc_hbm,
           as_v, ad_v, srcs_v, dsts_v, gbuf, sbuf, gsem, ssem, acc_s):
    c = lax.axis_index("c")
    s = lax.axis_index("s")
    wid = c * NTILE + s
    iota16 = lax.broadcasted_iota(i32, (16,), 0)
    c16 = jnp.full((16,), NCLASS, i32)
    pltpu.sync_copy(al2t_hbm.at[0], as_v)
    pltpu.sync_copy(al2t_hbm.at[1], ad_v)
    _zero_buf0(sbuf, WS2)
    _acc_chunks(
        s,
        lambda off: pltpu.sync_copy(sbuf.at[0], acc_s.at[pl.ds(off, G)]),
        lambda off: pltpu.sync_copy(sbuf.at[0, 0:TAIL],
                                    acc_s.at[pl.ds(off, TAIL)]))
    plsc.subcore_barrier()

    nch = NCHUNK // (NSC * NTILE)
    cbase = wid * nch

    def attend_scale(gb, sb, db, t, jg):
        ebase = jg * G
        for g in range(8):
            e16 = iota16 + (g * 16)
            idx_s = srcs_v[db, t, pl.ds(g * 16, 16)]
            idx_d = dsts_v[db, t, pl.ds(g * 16, 16)]
            a_s = plsc.load_gather(as_v, [idx_s])
            a_d = plsc.load_gather(ad_v, [idx_d])
            l = a_s + a_d
            w = jnp.exp(jnp.where(l > 0, l, ALPHA * l))
            w = jnp.where((ebase + g * 16 + iota16) < E, w, 0.0)
            plsc.store_scatter(sbuf.at[sb], [e16, c16], w)

        @pl.loop(0, G, unroll=8)
        def _(e):
            ef = jnp.full((16,), e, i32)
            w = plsc.load_gather(sbuf.at[sb], [ef, c16])
            sbuf[sb, e, pl.ds(0, 16)] = gbuf[gb, e, pl.ds(0, 16)] * w

    _edge_ring(nch, cbase, None, htbl2_hbm, src_hbm, dst_hbm,
               srcs_v, dsts_v, gbuf, sbuf, gsem, ssem, acc_s, attend_scale)
    plsc.subcore_barrier()
    _acc_chunks(
        s,
        lambda off: pltpu.sync_copy(acc_s.at[pl.ds(off, G)],
                                    acc_hbm.at[c, pl.ds(off, G)]),
        lambda off: pltpu.sync_copy(acc_s.at[pl.ds(off, TAIL)],
                                    acc_hbm.at[c, pl.ds(off, TAIL)]))


def _tc_final(acc_ref, out_ref):
    t = acc_ref[0] + acc_ref[1]
    h = t[:, 0:NCLASS] / t[:, NCLASS:NCLASS + 1]
    z = h - jnp.max(h, axis=1, keepdims=True)
    out_ref[...] = z - jnp.log(jnp.sum(jnp.exp(z), axis=1, keepdims=True))


def _sc_mesh():
    return plsc.VectorSubcoreMesh(core_axis_name="c", subcore_axis_name="s",
                                  num_cores=NSC, num_subcores=NTILE)


_SC_PARAMS = pltpu.CompilerParams(needs_layout_passes=False,
                                  use_tc_tiling_on_sc=False)


def kernel(features, edge_list, W_heads, b_heads, a_heads, W_out, b_out, a_out):
    W1 = W_heads.reshape(NHEAD * NHID, NFEAT).T
    asrc = a_heads[:, 0, :NHID]
    adst = a_heads[:, 0, NHID:]
    eye = jnp.eye(NHEAD, dtype=f32)
    A_as = (eye[:, None, :] * asrc[:, :, None]).reshape(NHEAD * NHID, NHEAD)
    A_ad = (eye[:, None, :] * adst[:, :, None]).reshape(NHEAD * NHID, NHEAD)
    Acat = jnp.concatenate([A_as, A_ad], axis=1)
    pad = E_PAD - E
    src2d = jnp.pad(edge_list[0], (0, pad)).reshape(NCHUNK, G)
    dst2d = jnp.pad(edge_list[1], (0, pad)).reshape(NCHUNK, G)

    h_all, al_nm = pl.pallas_call(
        _tc_prep1,
        grid=(N // TB,),
        in_specs=[pl.BlockSpec((TB, NFEAT), lambda i: (i, 0)),
                  pl.BlockSpec((NFEAT, 256), lambda i: (0, 0)),
                  pl.BlockSpec((256, 8), lambda i: (0, 0))],
        out_specs=[pl.BlockSpec((TB, 256), lambda i: (i, 0)),
                   pl.BlockSpec((TB, 8), lambda i: (i, 0))],
        out_shape=[jax.ShapeDtypeStruct((N, 256), f32),
                   jax.ShapeDtypeStruct((N, 8), f32)],
    )(features, W1, Acat)
    htbl2d = (h_all.reshape(N, NHEAD, 2, 2, 16)
              .transpose(1, 0, 2, 4, 3)
              .reshape(NHEAD * N, NHID).astype(bf16))
    alt = al_nm.T

    acc1 = pl.kernel(
        _sc_l1,
        out_type=jax.ShapeDtypeStruct((NHEAD, N, WS1), f32),
        mesh=_sc_mesh(),
        compiler_params=_SC_PARAMS,
        scratch_types=[
            pltpu.VMEM((N,), f32),
            pltpu.VMEM((N,), f32),
            pltpu.VMEM((2, SUP, G), i32),
            pltpu.VMEM((2, SUP, G), i32),
            pltpu.VMEM((NBUFG, G, NHID), bf16),
            pltpu.VMEM((NBUFS, G, WS1), f32),
            pltpu.SemaphoreType.DMA((NBUFG,)),
            pltpu.SemaphoreType.DMA((NBUFS,)),
            pltpu.VMEM_SHARED((N, WS1), f32),
        ],
    )(htbl2d, alt, src2d, dst2d)

    W2 = W_out.T
    a2cat = jnp.stack([a_out[0, :NCLASS], a_out[0, NCLASS:]], axis=1)
    htbl2, al2_nm = pl.pallas_call(
        _tc_prep2,
        grid=(N // TB,),
        in_specs=[pl.BlockSpec((NHEAD, TB, WS1), lambda i: (0, i, 0)),
                  pl.BlockSpec((256, NCLASS), lambda i: (0, 0)),
                  pl.BlockSpec((1, NCLASS), lambda i: (0, 0)),
                  pl.BlockSpec((NCLASS, 2), lambda i: (0, 0))],
        out_specs=[pl.BlockSpec((TB, NCLASS), lambda i: (i, 0)),
                   pl.BlockSpec((TB, 8), lambda i: (i, 0))],
        out_shape=[jax.ShapeDtypeStruct((N, NCLASS), f32),
                   jax.ShapeDtypeStruct((N, 8), f32)],
    )(acc1, W2, b_out.reshape(1, NCLASS), a2cat)
    al2t = al2_nm[:, 0:2].T

    acc2 = pl.kernel(
        _sc_l2,
        out_type=jax.ShapeDtypeStruct((NSC, N, WS2), f32),
        mesh=_sc_mesh(),
        compiler_params=_SC_PARAMS,
        scratch_types=[
            pltpu.VMEM((N,), f32),
            pltpu.VMEM((N,), f32),
            pltpu.VMEM((2, SUP, G), i32),
            pltpu.VMEM((2, SUP, G), i32),
            pltpu.VMEM((NBUFG, G, NCLASS), f32),
            pltpu.VMEM((NBUFS, G, WS2), f32),
            pltpu.SemaphoreType.DMA((NBUFG,)),
            pltpu.SemaphoreType.DMA((NBUFS,)),
            pltpu.VMEM_SHARED((N, WS2), f32),
        ],
    )(htbl2, al2t, src2d, dst2d)

    return pl.pallas_call(
        _tc_final,
        grid=(N // TB,),
        in_specs=[pl.BlockSpec((2, TB, WS2), lambda i: (0, i, 0))],
        out_specs=pl.BlockSpec((TB, NCLASS), lambda i: (i, 0)),
        out_shape=jax.ShapeDtypeStruct((N, NCLASS), f32),
    )(acc2)
